# Initial kernel scaffold; baseline (speedup 1.0000x reference)
import jax
import jax.numpy as jnp
from jax.experimental import pallas as pl


def kernel(xyz, atom_xyz, atomtypes, batch, atom_batch, tW1, tb1, tW2, tb2, aaW1, aab1, aaW2, aab2, aagw, aagb, emW1, emb1, emW2, emb2, emgw, emgb):
    raise NotImplementedError("write your pallas kernel here")


# speedup vs baseline: 4.4615x; 4.4615x over previous
import functools
import jax
import jax.numpy as jnp
from jax import lax
from jax.experimental import pallas as pl
from jax.experimental.pallas import tpu as pltpu
from jax.experimental.pallas import tpu_sc as plsc

F32 = jnp.float32
D = 16
H = 33
HP = 48
K = 16
NA, NAP = 5000, 5120
NP, NPP = 12000, 12288
NC, NS = 2, 16
NW = NC * NS


def _leaky(x):
    return jnp.where(x >= 0, x, 0.2 * x)


def _prep_body(x_ref, w1, b1, w2, b2, w1a, b1n, w1b, at_ref, a_ref, g_ref):
    x = x_ref[...]
    h = _leaky(jnp.dot(x, w1[...], preferred_element_type=F32) + b1[...])
    at = jnp.dot(h, w2[...], preferred_element_type=F32) + b2[...]
    at_ref[...] = at
    a_ref[...] = jnp.dot(at, w1a[...], preferred_element_type=F32) + b1n[...]
    g_ref[...] = jnp.dot(at, w1b[...], preferred_element_type=F32)


def _prep_call(atypes_p, w1, b1, w2, b2, w1a, b1n, w1b):
    BL = 512
    n = atypes_p.shape[0]
    row = lambda i: (i, 0)
    whole = lambda i: (0, 0)
    return pl.pallas_call(
        _prep_body,
        grid=(n // BL,),
        in_specs=[pl.BlockSpec((BL, D), row)] + [pl.BlockSpec(w.shape, whole) for w in (w1, b1, w2, b2, w1a, b1n, w1b)],
        out_specs=[pl.BlockSpec((BL, D), row), pl.BlockSpec((BL, HP), row), pl.BlockSpec((BL, HP), row)],
        out_shape=[jax.ShapeDtypeStruct((n, D), F32),
                   jax.ShapeDtypeStruct((n, HP), F32),
                   jax.ShapeDtypeStruct((n, HP), F32)],
    )(atypes_p, w1, b1, w2, b2, w1a, b1n, w1b)


def _knn_body(x_ref, yt_ref, idx_ref, dst_ref, *, mask_diag, qb, ncol):
    x = x_ref[...]
    yt = yt_ref[...]
    xn = jnp.sum(x * x, axis=1, keepdims=True)
    yn = jnp.sum(yt * yt, axis=0, keepdims=True)
    xy = jnp.dot(x, yt, preferred_element_type=F32,
                 precision=lax.Precision.HIGHEST)
    d = jnp.maximum(xn + yn - 2.0 * xy, 0.0)
    col = lax.broadcasted_iota(jnp.int32, (qb, ncol), 1)
    if mask_diag:
        row = pl.program_id(0) * qb + lax.broadcasted_iota(jnp.int32, (qb, ncol), 0)
        d = jnp.where(row == col, jnp.inf, d)
    idxs, vals = [], []
    for _ in range(K):
        m = jnp.min(d, axis=1)
        a = jnp.min(jnp.where(d == m[:, None], col, jnp.int32(1 << 30)), axis=1)
        idxs.append(a)
        vals.append(m)
        d = jnp.where(col == a[:, None], jnp.inf, d)
    idx_ref[...] = jnp.stack(idxs, axis=1)
    dst_ref[...] = jnp.stack(vals, axis=1)


def _knn_call(x_p, yt, mask_diag):
    QB = 256
    n, ncol = x_p.shape[0], yt.shape[1]
    body = functools.partial(_knn_body, mask_diag=mask_diag, qb=QB, ncol=ncol)
    return pl.pallas_call(
        body,
        grid=(n // QB,),
        in_specs=[pl.BlockSpec((QB, 3), lambda i: (i, 0)),
                  pl.BlockSpec((3, ncol), lambda i: (0, 0))],
        out_specs=[pl.BlockSpec((QB, K), lambda i: (i, 0)),
                   pl.BlockSpec((QB, K), lambda i: (i, 0))],
        out_shape=[jax.ShapeDtypeStruct((n, K), jnp.int32),
                   jax.ShapeDtypeStruct((n, K), F32)],
    )(x_p, yt)


def _make_gather(n_edges):
    CH = 128
    nper = n_edges // NW
    nch = nper // CH
    assert nper % CH == 0
    mesh = plsc.VectorSubcoreMesh(core_axis_name="c", subcore_axis_name="s")

    @functools.partial(
        pl.kernel,
        mesh=mesh,
        out_type=jax.ShapeDtypeStruct((n_edges, HP), F32),
        scratch_types=[pltpu.VMEM((CH,), jnp.int32),
                       pltpu.VMEM((CH, HP), F32),
                       pltpu.SemaphoreType.DMA],
        compiler_params=pltpu.CompilerParams(use_tc_tiling_on_sc=False),
    )
    def gather(table_hbm, idx_hbm, out_hbm, idx_v, rows_v, sem):
        wid = lax.axis_index("s") * NC + lax.axis_index("c")
        base = wid * nper

        def body(c, carry):
            off = base + c * CH
            pltpu.sync_copy(idx_hbm.at[pl.ds(off, CH)], idx_v)
            pltpu.async_copy(table_hbm.at[idx_v], rows_v, sem).wait()
            pltpu.sync_copy(rows_v, out_hbm.at[pl.ds(off, CH)])
            return carry

        lax.fori_loop(0, nch, body, 0)

    return gather


def _post_body(e_ref, a_ref, d_ref, prev_ref, wd_ref, w2_ref, b2k_ref, gw_ref,
               gb_ref, *proj_and_out, n_proj):
    proj_w = proj_and_out[:2 * n_proj:2]
    proj_b = proj_and_out[1:2 * n_proj:2]
    out_ref = proj_and_out[2 * n_proj]
    proj_refs = proj_and_out[2 * n_proj + 1:]

    a = a_ref[...]
    wd = wd_ref[...]
    s = jnp.zeros(a.shape, F32)
    for j in range(K):
        hj = a + e_ref[:, j, :] + d_ref[:, j][:, None] * wd
        s = s + _leaky(hj)
    msg = jnp.dot(s, w2_ref[...], preferred_element_type=F32) + b2k_ref[...]
    eps = 1e-5
    g0 = msg[:, 0:8]
    g1 = msg[:, 8:16]
    mu0 = jnp.mean(g0, axis=1, keepdims=True)
    mu1 = jnp.mean(g1, axis=1, keepdims=True)
    v0 = jnp.mean((g0 - mu0) ** 2, axis=1, keepdims=True)
    v1 = jnp.mean((g1 - mu1) ** 2, axis=1, keepdims=True)
    xn = jnp.concatenate([(g0 - mu0) / jnp.sqrt(v0 + eps),
                          (g1 - mu1) / jnp.sqrt(v1 + eps)], axis=1)
    out = prev_ref[...] + _leaky(xn * gw_ref[...] + gb_ref[...])
    out_ref[...] = out
    for wref, bref, pref in zip(proj_w, proj_b, proj_refs):
        pref[...] = jnp.dot(out, wref[...], preferred_element_type=F32) + bref[...]


def _post_call(e3, a, dist, prev, wd, w2p, b2k, gw, gb, projs):
    BL = 512
    n = a.shape[0]
    row = lambda i: (i, 0)
    whole = lambda i: (0, 0)
    n_proj = len(projs)
    proj_args = []
    proj_specs = []
    for (w, b) in projs:
        proj_args += [w, b]
        proj_specs += [pl.BlockSpec(w.shape, whole), pl.BlockSpec(b.shape, whole)]
    body = functools.partial(_post_body, n_proj=n_proj)
    out_specs = [pl.BlockSpec((BL, D), row)] + \
                [pl.BlockSpec((BL, w.shape[1]), row) for (w, _) in projs]
    out_shape = [jax.ShapeDtypeStruct((n, D), F32)] + \
                [jax.ShapeDtypeStruct((n, w.shape[1]), F32) for (w, _) in projs]
    res = pl.pallas_call(
        body,
        grid=(n // BL,),
        in_specs=[pl.BlockSpec((BL, K, HP), lambda i: (i, 0, 0)),
                  pl.BlockSpec((BL, HP), row),
                  pl.BlockSpec((BL, K), row),
                  pl.BlockSpec((BL, D), row),
                  pl.BlockSpec(wd.shape, whole),
                  pl.BlockSpec(w2p.shape, whole),
                  pl.BlockSpec(b2k.shape, whole),
                  pl.BlockSpec(gw.shape, whole),
                  pl.BlockSpec(gb.shape, whole)] + proj_specs,
        out_specs=out_specs,
        out_shape=out_shape,
    )(e3, a, dist, prev, wd, w2p, b2k, gw, gb, *proj_args)
    return res


def _pad_h(w):
    return jnp.pad(w, ((0, 0), (0, HP - w.shape[1])))


def kernel(xyz, atom_xyz, atomtypes, batch, atom_batch, tW1, tb1, tW2, tb2,
           aaW1, aab1, aaW2, aab2, aagw, aagb, emW1, emb1, emW2, emb2,
           emgw, emgb):
    atypes_p = jnp.pad(atomtypes, ((0, NAP - NA), (0, 0)))
    ax_p = jnp.pad(atom_xyz, ((0, NAP - NA), (0, 0)), constant_values=1e6)
    x_p = jnp.pad(xyz, ((0, NPP - NP), (0, 0)), constant_values=1e6)
    axT = jnp.transpose(ax_p)

    aaW1a = [_pad_h(aaW1[i][:D, :]) for i in range(3)]
    aaW1b = [_pad_h(aaW1[i][D:2 * D, :]) for i in range(3)]
    aawd = [_pad_h(aaW1[i][2 * D, :][None, :]) for i in range(3)]
    aab1p = [_pad_h(aab1[i][None, :]) for i in range(3)]
    aaW2p = [jnp.pad(aaW2[i], ((0, HP - H), (0, 0))) for i in range(3)]
    aab2k = [K * aab2[i][None, :] for i in range(3)]
    emW1a = [_pad_h(emW1[i][:D, :]) for i in range(3)]
    emW1b = [_pad_h(emW1[i][D:2 * D, :]) for i in range(3)]
    emwd = [_pad_h(emW1[i][2 * D, :][None, :]) for i in range(3)]
    emb1p = [_pad_h(emb1[i][None, :]) for i in range(3)]
    emW2p = [jnp.pad(emW2[i], ((0, HP - H), (0, 0))) for i in range(3)]
    emb2k = [K * emb2[i][None, :] for i in range(3)]
    emW1b_stack = jnp.concatenate(emW1b, axis=1)

    at, A, G = _prep_call(atypes_p, tW1, tb1[None, :], tW2, tb2[None, :],
                          aaW1a[0], aab1p[0], aaW1b[0])

    idxA, dA = _knn_call(ax_p, axT, mask_diag=True)
    idxP, dP = _knn_call(x_p, axT, mask_diag=False)
    idxA_flat = idxA.reshape(-1)
    idxP_flat = idxP.reshape(-1)

    gather_a = _make_gather(NAP * K)
    gather_p = _make_gather(NPP * K)

    out = at
    gem3 = None
    for i in range(3):
        e = gather_a(G, idxA_flat)
        e3 = e.reshape(NAP, K, HP)
        zb = jnp.zeros((1, HP), F32)
        if i < 2:
            projs = [(aaW1a[i + 1], aab1p[i + 1]), (aaW1b[i + 1], zb)]
            out, A, G = _post_call(e3, A, dA, out, aawd[i], aaW2p[i],
                                   aab2k[i], aagw[i][None, :], aagb[i][None, :], projs)
        else:
            projs = [(emW1b_stack, jnp.zeros((1, 3 * HP), F32))]
            out, gem3 = _post_call(e3, A, dA, out, aawd[i], aaW2p[i],
                                   aab2k[i], aagw[i][None, :], aagb[i][None, :], projs)
    gem = [gem3[:, j * HP:(j + 1) * HP] for j in range(3)]

    emb = jnp.ones((NPP, D), F32)
    a0_row = jnp.sum(emW1a[0], axis=0, keepdims=True) + emb1p[0]
    A = jnp.tile(a0_row, (NPP, 1))
    for i in range(3):
        e = gather_p(gem[i], idxP_flat)
        e3 = e.reshape(NPP, K, HP)
        projs = [(emW1a[i + 1], emb1p[i + 1])] if i < 2 else []
        res = _post_call(e3, A, dP, emb, emwd[i], emW2p[i], emb2k[i],
                         emgw[i][None, :], emgb[i][None, :], projs)
        emb = res[0]
        if i < 2:
            A = res[1]
    return emb[:NP]
